# Initial kernel scaffold; baseline (speedup 1.0000x reference)
import jax
import jax.numpy as jnp
from jax.experimental import pallas as pl


def kernel(inputs, initial_hidden_state, supports, Wg0, bg0, Wc0, bc0, Wg1, bg1, Wc1, bc1):
    raise NotImplementedError("write your pallas kernel here")


# speedup vs baseline: 7.4307x; 7.4307x over previous
import jax
import jax.numpy as jnp
from jax.experimental import pallas as pl
from jax.experimental.pallas import tpu as pltpu
from functools import partial

T, B, N, D_IN = 12, 64, 207, 2
HID = 64
P = 2
BP = B // P

_DIFF_DIMS = (((1,), (0,)), ((), ()))
_PROJ_DIMS = (((2,), (0,)), ((), ()))


def _pack(a):
    *lead, b, n, f = a.shape
    a = jnp.moveaxis(a, -2, -3)
    return a.reshape(*lead, n, b // P, P * f)


def _split_weights(W, fin, out):
    Wr = W.reshape(fin, 3, out)
    mats = (Wr[:, 0, :] - Wr[:, 2, :], Wr[:, 1, :], 2.0 * Wr[:, 2, :])
    eye = jnp.eye(P, dtype=W.dtype)
    return [jnp.kron(eye, m) for m in mats]


def _layer_kernel(x_ref, h0_ref, s_ref, wg_ref, bg_ref, wc_ref, bc_ref,
                  out_ref, h_scr):
    t = pl.program_id(0)

    @pl.when(t == 0)
    def _():
        h_scr[...] = h0_ref[...]

    S = s_ref[...]
    x = x_ref[0]
    h = h_scr[...]

    diff = partial(jax.lax.dot_general, dimension_numbers=_DIFF_DIMS,
                   preferred_element_type=jnp.float32)
    proj = partial(jax.lax.dot_general, dimension_numbers=_PROJ_DIMS,
                   preferred_element_type=jnp.float32)

    x1 = diff(S, x)
    x2 = diff(S, x1)
    h1 = diff(S, h)
    h2 = diff(S, h1)

    pfin = x.shape[-1]

    def gconv(w_ref, b_ref, hs):
        w = w_ref[...]
        acc = b_ref[...]
        for m, (xv, hv) in enumerate(zip((x, x1, x2), hs)):
            wx = w[m * (pfin + P * HID):m * (pfin + P * HID) + pfin]
            wh = w[m * (pfin + P * HID) + pfin:(m + 1) * (pfin + P * HID)]
            acc = acc + proj(xv, wx) + proj(hv, wh)
        return acc

    g = gconv(wg_ref, bg_ref, (h, h1, h2))
    v = jax.nn.sigmoid(g)
    r = v[:, :, :P * HID]
    u = v[:, :, P * HID:]

    rs = r * h
    rs1 = diff(S, rs)
    rs2 = diff(S, rs1)

    c = gconv(wc_ref, bc_ref, (rs, rs1, rs2))
    c = jnp.tanh(c)

    hnew = u * h + (1.0 - u) * c
    h_scr[...] = hnew
    out_ref[0] = hnew


def _run_layer(xseq, h0, S, Wg, bg, Wc, bc, fin):
    def stack_w(W, out):
        mats = _split_weights(W, fin + HID, out)
        rows = []
        for m in mats:
            m3 = m.reshape(P, fin + HID, P * out)
            rows.append(jnp.concatenate(
                [m3[:, :fin].reshape(P * fin, P * out),
                 m3[:, fin:].reshape(P * HID, P * out)], axis=0))
        return jnp.concatenate(rows, axis=0)

    wg = stack_w(Wg, 2 * HID)
    wc = stack_w(Wc, HID)
    bgp = jnp.tile(bg, (P,))
    bcp = jnp.tile(bc, (P,))
    cols = jnp.arange(P * 2 * HID).reshape(P, 2, HID)
    perm = jnp.moveaxis(cols, 1, 0).reshape(-1)
    wg = wg[:, perm]
    bgp = bgp[perm]

    full = lambda shape: pl.BlockSpec(shape, lambda t: (0,) * len(shape))
    seq = lambda shape: pl.BlockSpec(shape, lambda t: (t, 0, 0, 0))

    out = pl.pallas_call(
        _layer_kernel,
        grid=(T,),
        in_specs=[
            seq((1, N, BP, P * fin)),
            full((N, BP, P * HID)),
            full((N, N)),
            full(wg.shape),
            full(bgp.shape),
            full(wc.shape),
            full(bcp.shape),
        ],
        out_specs=seq((1, N, BP, P * HID)),
        out_shape=jax.ShapeDtypeStruct((T, N, BP, P * HID), jnp.float32),
        scratch_shapes=[pltpu.VMEM((N, BP, P * HID), jnp.float32)],
    )(xseq, h0, S, wg, bgp, wc, bcp)
    return out


@jax.jit
def kernel(inputs, initial_hidden_state, supports, Wg0, bg0, Wc0, bc0,
           Wg1, bg1, Wc1, bc1):
    S = supports[0]
    xseq = _pack(inputs)
    h_init = _pack(initial_hidden_state.reshape(2, B, N, HID))

    cur0 = _run_layer(xseq, h_init[0], S, Wg0, bg0, Wc0, bc0, D_IN)
    cur1 = _run_layer(cur0, h_init[1], S, Wg1, bg1, Wc1, bc1, HID)

    def to_ref(a):
        *lead, n, bp, pf = a.shape
        a = a.reshape(*lead, n, bp * P, HID)
        a = jnp.moveaxis(a, -3, -2)
        return a.reshape(*lead, B, N * HID)

    current = to_ref(cur1)
    output_hidden = jnp.stack([to_ref(cur0[-1]), to_ref(cur1[-1])], axis=0)
    return (output_hidden, current)
